# Initial kernel scaffold; baseline (speedup 1.0000x reference)
import jax
import jax.numpy as jnp
from jax.experimental import pallas as pl


def kernel(inputs):
    raise NotImplementedError("write your pallas kernel here")


# speedup vs baseline: 70.2742x; 70.2742x over previous
import functools

import jax
import jax.numpy as jnp
from jax import lax
from jax.experimental import pallas as pl
from jax.experimental.pallas import tpu as pltpu
from jax.experimental.pallas import tpu_sc as plsc

K = 8
B = 32
H = 32768
C = 64
L = 16
NG = C // L
CH = 256
NCHUNK = H // CH
RB = 8
NBLK = CH // RB


def _neg_inf():
    return jnp.full((L,), -jnp.inf, dtype=jnp.float32)


def _kmax_body(x_hbm, out_hbm, buf, state, sem0, sem1):
    cid = lax.axis_index("c")
    sid = lax.axis_index("s")
    b = sid * 2 + cid

    for i in range(K):
        for g in range(NG):
            state[i, pl.ds(g * L, L)] = _neg_inf()

    def start_dma(chunk, slot, sem):
        pltpu.make_async_copy(
            x_hbm.at[b, pl.ds(chunk * CH, CH)], buf.at[slot], sem
        ).start()

    def wait_dma(chunk, slot, sem):
        pltpu.make_async_copy(
            x_hbm.at[b, pl.ds(chunk * CH, CH)], buf.at[slot], sem
        ).wait()

    start_dma(0, 0, sem0)
    start_dma(1, 1, sem1)

    def process_chunk(slot, r7s):
        def block(j, r7s_in):
            base = j * RB
            maccs = [buf[slot, base, pl.ds(g * L, L)] for g in range(NG)]
            for kk in range(1, RB):
                for g in range(NG):
                    maccs[g] = jnp.maximum(
                        maccs[g], buf[slot, base + kk, pl.ds(g * L, L)]
                    )
            ms = [maccs[g] > r7s_in[g] for g in range(NG)]
            hit = jnp.any((ms[0] | ms[1]) | (ms[2] | ms[3]))

            def slow(rs):
                outs = []
                for g in range(NG):
                    def insert(g=g):
                        s = [state[i, pl.ds(g * L, L)] for i in range(K)]
                        for kk in range(RB):
                            t = buf[slot, base + kk, pl.ds(g * L, L)]
                            for i in range(K):
                                hi = jnp.maximum(s[i], t)
                                t = jnp.minimum(s[i], t)
                                s[i] = hi
                        for i in range(K):
                            state[i, pl.ds(g * L, L)] = s[i]
                        return s[K - 1]

                    outs.append(
                        lax.cond(jnp.any(ms[g]), insert, lambda g=g: rs[g])
                    )
                return tuple(outs)

            return lax.cond(hit, slow, lambda rs: rs, r7s_in)

        return lax.fori_loop(0, NBLK, block, r7s)

    def outer(i, r7s):
        c0 = 2 * i
        wait_dma(c0, 0, sem0)
        r7s = process_chunk(0, r7s)

        @pl.when(c0 + 2 < NCHUNK)
        def _():
            start_dma(c0 + 2, 0, sem0)

        wait_dma(c0 + 1, 1, sem1)
        r7s = process_chunk(1, r7s)

        @pl.when(c0 + 3 < NCHUNK)
        def _():
            start_dma(c0 + 3, 1, sem1)

        return r7s

    r7s = (_neg_inf(), _neg_inf(), _neg_inf(), _neg_inf())
    lax.fori_loop(0, NCHUNK // 2, outer, r7s)

    pltpu.sync_copy(state, out_hbm.at[b])


@jax.jit
def _kmax(x):
    mesh = plsc.VectorSubcoreMesh(core_axis_name="c", subcore_axis_name="s")
    f = functools.partial(
        pl.kernel,
        out_type=jax.ShapeDtypeStruct((B, K, C), jnp.float32),
        mesh=mesh,
        compiler_params=pltpu.CompilerParams(needs_layout_passes=False),
        scratch_types=[
            pltpu.VMEM((2, CH, C), jnp.float32),
            pltpu.VMEM((K, C), jnp.float32),
            pltpu.SemaphoreType.DMA,
            pltpu.SemaphoreType.DMA,
        ],
    )(_kmax_body)
    return f(x)


def kernel(inputs):
    x = inputs.reshape(B, H, C)
    out = _kmax(x)
    return out.reshape(B, K, 1, C)
